# baseline (speedup 1.0000x reference)
import functools

import jax
import jax.numpy as jnp
from jax import lax
from jax.experimental import pallas as pl
from jax.experimental.pallas import tpu as pltpu
from jax.experimental.pallas import tpu_sc as plsc

E = 8
TOP_K = 2
N_GROUP = 4
TOPK_GROUP = 2
D_MODEL = 1024
D_FF = 768
T = 2048

_NEG = -1e30

_B = 256
_NB = (T * TOP_K) // _B + E
_P = _NB * _B

_RB = 256
_NRB = T // _RB


def _perm_mat():
    j = lax.broadcasted_iota(jnp.int32, (E, E), 0)
    e = lax.broadcasted_iota(jnp.int32, (E, E), 1)
    return j, e


def _rot(s, d):
    return jnp.concatenate([s[:, d:], s[:, :d]], axis=1)


def _combine_block(x, gate_w, e_bias):
    lane = lax.broadcasted_iota(jnp.int32, (1, E), 1)
    ties = {d: jnp.where((lane + d) % E < lane, 1.0, 0.0) for d in range(1, E)}

    logits = lax.dot_general(
        x, gate_w, (((1,), (1,)), ((), ())),
        preferred_element_type=jnp.float32)
    scores = 1.0 / (1.0 + jnp.exp(-logits))
    sfc = scores + e_bias

    even = (lane % 2) == 0
    gs = sfc + jnp.where(even, _rot(sfc, 1), _rot(sfc, E - 1))
    grank = jnp.zeros_like(gs)
    g_of = lane // 2
    for dg in range(1, N_GROUP):
        grot = _rot(gs, 2 * dg)
        gtie = jnp.where((g_of + dg) % N_GROUP < g_of, 1.0, 0.0)
        beats = jnp.where(grot > gs, 1.0, 0.0)
        tiebk = jnp.where(grot == gs, gtie, 0.0)
        grank = grank + jnp.maximum(beats, tiebk)
    masked = jnp.where(grank < TOPK_GROUP, sfc, _NEG)

    rank = jnp.zeros_like(masked)
    for d in range(1, E):
        srot = _rot(masked, d)
        beats = jnp.where(srot > masked, 1.0, 0.0)
        tiebk = jnp.where(srot == masked, ties[d], 0.0)
        rank = rank + jnp.maximum(beats, tiebk)
    selm = jnp.where(rank < TOP_K, 1.0, 0.0)
    w_raw = selm * scores
    denom = jnp.sum(w_raw, axis=1, keepdims=True) + 1e-20
    return w_raw / denom


def _frontend_kernel(x_ref, gw_ref, eb_ref,
                     pos0_ref, pos1_ref, w0_ref, w1_ref, be_ref,
                     comb_s, ranks_s, ltri_s, carry):
    i = pl.program_id(0)
    j8, e8 = _perm_mat()

    @pl.when(i == 0)
    def _():
        r0 = lax.broadcasted_iota(jnp.int32, (_RB, _RB), 0)
        r1 = lax.broadcasted_iota(jnp.int32, (_RB, _RB), 1)
        ltri_s[...] = jnp.where(r0 > r1, 1.0, 0.0)
        carry[...] = jnp.zeros_like(carry)

    @pl.when(i < _NRB)
    def _():
        off = pl.multiple_of(i * _RB, _RB)
        combine = _combine_block(x_ref[...], gw_ref[...], eb_ref[...])
        comb_s[pl.ds(off, _RB), :] = combine
        sel = jnp.where(combine > 0.0, 1.0, 0.0)
        ranks = jnp.dot(ltri_s[...], sel, preferred_element_type=jnp.float32)
        ranks_s[pl.ds(off, _RB), :] = ranks + carry[...]
        carry[...] = carry[...] + jnp.sum(sel, axis=0, keepdims=True)

    @pl.when(i >= _NRB)
    def _():
        off = pl.multiple_of((i - _NRB) * _RB, _RB)
        counts = carry[...].astype(jnp.int32)
        padded = (((counts + (_B - 1)) // _B) * _B).astype(jnp.float32)
        mlt = jnp.where(j8 < e8, 1.0, 0.0)
        po = jnp.dot(padded, mlt, preferred_element_type=jnp.float32)
        mle = jnp.where(j8 <= e8, 1.0, 0.0)
        c = comb_s[pl.ds(off, _RB), :]
        sel = jnp.where(c > 0.0, 1.0, 0.0)
        cum = jnp.dot(sel, mle, preferred_element_type=jnp.float32)
        first = sel * jnp.where(cum == 1.0, 1.0, 0.0)
        second = sel * jnp.where(cum == 2.0, 1.0, 0.0)
        posmat = ranks_s[pl.ds(off, _RB), :] + po
        pos0_ref[...] = jnp.sum(posmat * first, axis=1,
                                keepdims=True).astype(jnp.int32)
        pos1_ref[...] = jnp.sum(posmat * second, axis=1,
                                keepdims=True).astype(jnp.int32)
        ones16 = jnp.ones((1, 16), jnp.float32)
        w0_ref[...] = jnp.sum(c * first, axis=1, keepdims=True) * ones16
        w1_ref[...] = jnp.sum(c * second, axis=1, keepdims=True) * ones16

        @pl.when(i == _NRB)
        def _():
            ends = po + padded
            total_end = ends[:, E - 1:E]
            be_cols = []
            for b in range(_NB):
                nb_before = jnp.sum(
                    jnp.where(ends <= float(b * _B), 1.0, 0.0),
                    axis=1, keepdims=True).astype(jnp.int32)
                valid = (b * _B) < total_end
                be_cols.append(jnp.where(valid, nb_before, -1))
            be_ref[...] = jnp.concatenate(be_cols, axis=1)


def _frontend(x, gate_w, e_bias):
    return pl.pallas_call(
        _frontend_kernel,
        grid=(2 * _NRB,),
        in_specs=[
            pl.BlockSpec((_RB, D_MODEL),
                         lambda i: (jnp.minimum(i, _NRB - 1), 0)),
            pl.BlockSpec((E, D_MODEL), lambda i: (0, 0)),
            pl.BlockSpec((1, E), lambda i: (0, 0)),
        ],
        out_specs=[
            pl.BlockSpec((_RB, 1), lambda i: (jnp.maximum(i - _NRB, 0), 0)),
            pl.BlockSpec((_RB, 1), lambda i: (jnp.maximum(i - _NRB, 0), 0)),
            pl.BlockSpec((_RB, 16), lambda i: (jnp.maximum(i - _NRB, 0), 0)),
            pl.BlockSpec((_RB, 16), lambda i: (jnp.maximum(i - _NRB, 0), 0)),
            pl.BlockSpec((1, _NB), lambda i: (0, 0)),
        ],
        out_shape=[
            jax.ShapeDtypeStruct((T, 1), jnp.int32),
            jax.ShapeDtypeStruct((T, 1), jnp.int32),
            jax.ShapeDtypeStruct((T, 16), jnp.float32),
            jax.ShapeDtypeStruct((T, 16), jnp.float32),
            jax.ShapeDtypeStruct((1, _NB), jnp.int32),
        ],
        scratch_shapes=[
            pltpu.VMEM((T, E), jnp.float32),
            pltpu.VMEM((T, E), jnp.float32),
            pltpu.VMEM((_RB, _RB), jnp.float32),
            pltpu.VMEM((1, E), jnp.float32),
        ],
        compiler_params=pltpu.CompilerParams(
            dimension_semantics=("arbitrary",)),
    )(x, gate_w, e_bias.reshape(1, E))


_NW = 32
_TPW = T // _NW


def _sc_scatter_body(x_hbm, p0_hbm, p1_hbm, xs_hbm,
                     idx0_v, idx1_v, rows_v, sem0, sem1):
    wid = lax.axis_index("s") * 2 + lax.axis_index("c")
    base = wid * _TPW
    pltpu.sync_copy(x_hbm.at[pl.ds(base, _TPW)], rows_v)
    pltpu.sync_copy(p0_hbm.at[pl.ds(base, _TPW)], idx0_v)
    pltpu.sync_copy(p1_hbm.at[pl.ds(base, _TPW)], idx1_v)
    cp0 = pltpu.async_copy(rows_v, xs_hbm.at[idx0_v], sem0)
    cp1 = pltpu.async_copy(rows_v, xs_hbm.at[idx1_v], sem1)
    cp0.wait()
    cp1.wait()


def _sc_scatter(x, pos0, pos1):
    mesh = plsc.VectorSubcoreMesh(core_axis_name="c", subcore_axis_name="s")
    kfn = functools.partial(
        pl.kernel,
        mesh=mesh,
        out_type=jax.ShapeDtypeStruct((_P, D_MODEL), jnp.float32),
        scratch_types=[
            pltpu.VMEM((_TPW,), jnp.int32),
            pltpu.VMEM((_TPW,), jnp.int32),
            pltpu.VMEM((_TPW, D_MODEL), jnp.float32),
            pltpu.SemaphoreType.DMA,
            pltpu.SemaphoreType.DMA,
        ],
    )(_sc_scatter_body)
    return kfn(x, pos0, pos1)


def _ffn_kernel(be_ref, xs_ref, wg_ref, wu_ref, wd_ref, y_ref,
                wg_s, wu_s, wd_s):
    b = pl.program_id(0)
    be = be_ref[b]
    prev = be_ref[jnp.maximum(b - 1, 0)]
    new_run = jnp.logical_or(b == 0, be != prev)

    @pl.when(jnp.logical_and(be >= 0, new_run))
    def _():
        wg_s[...] = wg_ref[0].astype(jnp.bfloat16)
        wu_s[...] = wu_ref[0].astype(jnp.bfloat16)
        wd_s[...] = wd_ref[0].astype(jnp.bfloat16)

    @pl.when(be >= 0)
    def _():
        x = xs_ref[...].astype(jnp.bfloat16)
        g = jnp.dot(x, wg_s[...], preferred_element_type=jnp.float32)
        u = jnp.dot(x, wu_s[...], preferred_element_type=jnp.float32)
        h = ((g / (1.0 + jnp.exp(-g))) * u).astype(jnp.bfloat16)
        y_ref[...] = jnp.dot(h, wd_s[...], preferred_element_type=jnp.float32)


def _ffn(be, xs, w_gate, w_up, w_down):
    grid_spec = pltpu.PrefetchScalarGridSpec(
        num_scalar_prefetch=1,
        grid=(_NB,),
        in_specs=[
            pl.BlockSpec((_B, D_MODEL), lambda b, be: (b, 0)),
            pl.BlockSpec((1, D_MODEL, D_FF),
                         lambda b, be: (jnp.maximum(be[b], 0), 0, 0)),
            pl.BlockSpec((1, D_MODEL, D_FF),
                         lambda b, be: (jnp.maximum(be[b], 0), 0, 0)),
            pl.BlockSpec((1, D_FF, D_MODEL),
                         lambda b, be: (jnp.maximum(be[b], 0), 0, 0)),
        ],
        out_specs=pl.BlockSpec((_B, D_MODEL), lambda b, be: (b, 0)),
        scratch_shapes=[
            pltpu.VMEM((D_MODEL, D_FF), jnp.bfloat16),
            pltpu.VMEM((D_MODEL, D_FF), jnp.bfloat16),
            pltpu.VMEM((D_FF, D_MODEL), jnp.bfloat16),
        ],
    )
    return pl.pallas_call(
        _ffn_kernel,
        grid_spec=grid_spec,
        out_shape=jax.ShapeDtypeStruct((_P, D_MODEL), jnp.float32),
        compiler_params=pltpu.CompilerParams(
            dimension_semantics=("arbitrary",)),
    )(be, xs, w_gate, w_up, w_down)


_CH = 32


def _sc_combine_body(y_hbm, p0_hbm, p1_hbm, w0_hbm, w1_hbm, out_hbm,
                     idx0_v, idx1_v, w0_v, w1_v, buf0_v, buf1_v, sem0, sem1):
    wid = lax.axis_index("s") * 2 + lax.axis_index("c")
    base = wid * _TPW
    for ck in range(_TPW // _CH):
        off = base + ck * _CH
        pltpu.sync_copy(p0_hbm.at[pl.ds(off, _CH)], idx0_v)
        pltpu.sync_copy(p1_hbm.at[pl.ds(off, _CH)], idx1_v)
        pltpu.sync_copy(w0_hbm.at[pl.ds(off, _CH)], w0_v)
        pltpu.sync_copy(w1_hbm.at[pl.ds(off, _CH)], w1_v)
        cp0 = pltpu.async_copy(y_hbm.at[idx0_v], buf0_v, sem0)
        cp1 = pltpu.async_copy(y_hbm.at[idx1_v], buf1_v, sem1)
        cp0.wait()
        cp1.wait()
        for r in range(_CH):
            wv0 = w0_v[r, :]
            wv1 = w1_v[r, :]

            def _row_fma(jj, _, r=r, wv0=wv0, wv1=wv1):
                o = jj * 16
                buf0_v[r, pl.ds(o, 16)] = (
                    buf0_v[r, pl.ds(o, 16)] * wv0
                    + buf1_v[r, pl.ds(o, 16)] * wv1)
                return 0
            lax.fori_loop(0, D_MODEL // 16, _row_fma, 0, unroll=8)
        pltpu.sync_copy(buf0_v, out_hbm.at[pl.ds(off, _CH)])


def _sc_combine(y, pos0, pos1, w0, w1):
    mesh = plsc.VectorSubcoreMesh(core_axis_name="c", subcore_axis_name="s")
    kfn = functools.partial(
        pl.kernel,
        mesh=mesh,
        out_type=jax.ShapeDtypeStruct((T, D_MODEL), jnp.float32),
        scratch_types=[
            pltpu.VMEM((_CH,), jnp.int32),
            pltpu.VMEM((_CH,), jnp.int32),
            pltpu.VMEM((_CH, 16), jnp.float32),
            pltpu.VMEM((_CH, 16), jnp.float32),
            pltpu.VMEM((_CH, D_MODEL), jnp.float32),
            pltpu.VMEM((_CH, D_MODEL), jnp.float32),
            pltpu.SemaphoreType.DMA,
            pltpu.SemaphoreType.DMA,
        ],
    )(_sc_combine_body)
    return kfn(y, pos0, pos1, w0, w1)


@jax.jit
def kernel(hidden_states, gate_w, e_bias, w_gate, w_up, w_down):
    x = hidden_states.reshape(-1, D_MODEL)
    pos0, pos1, w0, w1, be = _frontend(x, gate_w, e_bias)
    p0 = pos0.reshape(T)
    p1 = pos1.reshape(T)
    xs = _sc_scatter(x, p0, p1)
    y = _ffn(be.reshape(_NB), xs, w_gate, w_up, w_down)
    return _sc_combine(y, p0, p1, w0, w1)


# speedup vs baseline: 1.0400x; 1.0400x over previous
import functools

import jax
import jax.numpy as jnp
from jax import lax
from jax.experimental import pallas as pl
from jax.experimental.pallas import tpu as pltpu
from jax.experimental.pallas import tpu_sc as plsc

E = 8
TOP_K = 2
N_GROUP = 4
TOPK_GROUP = 2
D_MODEL = 1024
D_FF = 768
T = 2048

_NEG = -1e30

_B = 256
_NB = (T * TOP_K) // _B + E
_P = _NB * _B

_RB = 256
_NRB = T // _RB


def _perm_mat():
    j = lax.broadcasted_iota(jnp.int32, (E, E), 0)
    e = lax.broadcasted_iota(jnp.int32, (E, E), 1)
    return j, e


def _rot(s, d):
    return jnp.concatenate([s[:, d:], s[:, :d]], axis=1)


def _combine_block(x, gate_w, e_bias):
    lane = lax.broadcasted_iota(jnp.int32, (1, E), 1)
    ties = {d: jnp.where((lane + d) % E < lane, 1.0, 0.0) for d in range(1, E)}

    logits = lax.dot_general(
        x, gate_w, (((1,), (1,)), ((), ())),
        preferred_element_type=jnp.float32)
    scores = 1.0 / (1.0 + jnp.exp(-logits))
    sfc = scores + e_bias

    even = (lane % 2) == 0
    gs = sfc + jnp.where(even, _rot(sfc, 1), _rot(sfc, E - 1))
    grank = jnp.zeros_like(gs)
    g_of = lane // 2
    for dg in range(1, N_GROUP):
        grot = _rot(gs, 2 * dg)
        gtie = jnp.where((g_of + dg) % N_GROUP < g_of, 1.0, 0.0)
        beats = jnp.where(grot > gs, 1.0, 0.0)
        tiebk = jnp.where(grot == gs, gtie, 0.0)
        grank = grank + jnp.maximum(beats, tiebk)
    masked = jnp.where(grank < TOPK_GROUP, sfc, _NEG)

    rank = jnp.zeros_like(masked)
    for d in range(1, E):
        srot = _rot(masked, d)
        beats = jnp.where(srot > masked, 1.0, 0.0)
        tiebk = jnp.where(srot == masked, ties[d], 0.0)
        rank = rank + jnp.maximum(beats, tiebk)
    selm = jnp.where(rank < TOP_K, 1.0, 0.0)
    w_raw = selm * scores
    denom = jnp.sum(w_raw, axis=1, keepdims=True) + 1e-20
    return w_raw / denom


def _frontend_kernel(x_ref, gw_ref, eb_ref,
                     pos0_ref, pos1_ref, w0_ref, w1_ref, be_ref,
                     comb_s, ranks_s, ltri_s, carry):
    i = pl.program_id(0)
    j8, e8 = _perm_mat()

    @pl.when(i == 0)
    def _():
        r0 = lax.broadcasted_iota(jnp.int32, (_RB, _RB), 0)
        r1 = lax.broadcasted_iota(jnp.int32, (_RB, _RB), 1)
        ltri_s[...] = jnp.where(r0 > r1, 1.0, 0.0)
        carry[...] = jnp.zeros_like(carry)

    @pl.when(i < _NRB)
    def _():
        off = pl.multiple_of(i * _RB, _RB)
        combine = _combine_block(x_ref[...], gw_ref[...], eb_ref[...])
        comb_s[pl.ds(off, _RB), :] = combine
        sel = jnp.where(combine > 0.0, 1.0, 0.0)
        ranks = jnp.dot(ltri_s[...], sel, preferred_element_type=jnp.float32)
        ranks_s[pl.ds(off, _RB), :] = ranks + carry[...]
        carry[...] = carry[...] + jnp.sum(sel, axis=0, keepdims=True)

    @pl.when(i >= _NRB)
    def _():
        off = pl.multiple_of((i - _NRB) * _RB, _RB)
        counts = carry[...].astype(jnp.int32)
        padded = (((counts + (_B - 1)) // _B) * _B).astype(jnp.float32)
        mlt = jnp.where(j8 < e8, 1.0, 0.0)
        po = jnp.dot(padded, mlt, preferred_element_type=jnp.float32)
        mle = jnp.where(j8 <= e8, 1.0, 0.0)
        c = comb_s[pl.ds(off, _RB), :]
        sel = jnp.where(c > 0.0, 1.0, 0.0)
        cum = jnp.dot(sel, mle, preferred_element_type=jnp.float32)
        first = sel * jnp.where(cum == 1.0, 1.0, 0.0)
        second = sel * jnp.where(cum == 2.0, 1.0, 0.0)
        posmat = ranks_s[pl.ds(off, _RB), :] + po
        pos0_ref[...] = jnp.sum(posmat * first, axis=1,
                                keepdims=True).astype(jnp.int32)
        pos1_ref[...] = jnp.sum(posmat * second, axis=1,
                                keepdims=True).astype(jnp.int32)
        ones16 = jnp.ones((1, 16), jnp.float32)
        w0_ref[...] = jnp.sum(c * first, axis=1, keepdims=True) * ones16
        w1_ref[...] = jnp.sum(c * second, axis=1, keepdims=True) * ones16

        @pl.when(i == _NRB)
        def _():
            ends = po + padded
            total_end = ends[:, E - 1:E]
            be_cols = []
            for b in range(_NB):
                nb_before = jnp.sum(
                    jnp.where(ends <= float(b * _B), 1.0, 0.0),
                    axis=1, keepdims=True).astype(jnp.int32)
                valid = (b * _B) < total_end
                be_cols.append(jnp.where(valid, nb_before, -1))
            be_ref[...] = jnp.concatenate(be_cols, axis=1)


def _frontend(x, gate_w, e_bias):
    return pl.pallas_call(
        _frontend_kernel,
        grid=(2 * _NRB,),
        in_specs=[
            pl.BlockSpec((_RB, D_MODEL),
                         lambda i: (jnp.minimum(i, _NRB - 1), 0)),
            pl.BlockSpec((E, D_MODEL), lambda i: (0, 0)),
            pl.BlockSpec((1, E), lambda i: (0, 0)),
        ],
        out_specs=[
            pl.BlockSpec((_RB, 1), lambda i: (jnp.maximum(i - _NRB, 0), 0)),
            pl.BlockSpec((_RB, 1), lambda i: (jnp.maximum(i - _NRB, 0), 0)),
            pl.BlockSpec((_RB, 16), lambda i: (jnp.maximum(i - _NRB, 0), 0)),
            pl.BlockSpec((_RB, 16), lambda i: (jnp.maximum(i - _NRB, 0), 0)),
            pl.BlockSpec((1, _NB), lambda i: (0, 0)),
        ],
        out_shape=[
            jax.ShapeDtypeStruct((T, 1), jnp.int32),
            jax.ShapeDtypeStruct((T, 1), jnp.int32),
            jax.ShapeDtypeStruct((T, 16), jnp.float32),
            jax.ShapeDtypeStruct((T, 16), jnp.float32),
            jax.ShapeDtypeStruct((1, _NB), jnp.int32),
        ],
        scratch_shapes=[
            pltpu.VMEM((T, E), jnp.float32),
            pltpu.VMEM((T, E), jnp.float32),
            pltpu.VMEM((_RB, _RB), jnp.float32),
            pltpu.VMEM((1, E), jnp.float32),
        ],
        compiler_params=pltpu.CompilerParams(
            dimension_semantics=("arbitrary",)),
    )(x, gate_w, e_bias.reshape(1, E))


_NW = 32
_TPW = T // _NW


def _sc_scatter_body(x_hbm, p0_hbm, p1_hbm, xs_hbm,
                     idx0_v, idx1_v, rows_v, sem0, sem1):
    wid = lax.axis_index("s") * 2 + lax.axis_index("c")
    base = wid * _TPW
    ld0 = pltpu.async_copy(x_hbm.at[pl.ds(base, _TPW)], rows_v, sem0)
    ld1 = pltpu.async_copy(p0_hbm.at[pl.ds(base, _TPW)], idx0_v, sem1)
    ld2 = pltpu.async_copy(p1_hbm.at[pl.ds(base, _TPW)], idx1_v, sem1)
    ld0.wait()
    ld1.wait()
    ld2.wait()
    cp0 = pltpu.async_copy(rows_v, xs_hbm.at[idx0_v], sem0)
    cp1 = pltpu.async_copy(rows_v, xs_hbm.at[idx1_v], sem1)
    cp0.wait()
    cp1.wait()


def _sc_scatter(x, pos0, pos1):
    mesh = plsc.VectorSubcoreMesh(core_axis_name="c", subcore_axis_name="s")
    kfn = functools.partial(
        pl.kernel,
        mesh=mesh,
        out_type=jax.ShapeDtypeStruct((_P, D_MODEL), jnp.float32),
        scratch_types=[
            pltpu.VMEM((_TPW,), jnp.int32),
            pltpu.VMEM((_TPW,), jnp.int32),
            pltpu.VMEM((_TPW, D_MODEL), jnp.float32),
            pltpu.SemaphoreType.DMA,
            pltpu.SemaphoreType.DMA,
        ],
    )(_sc_scatter_body)
    return kfn(x, pos0, pos1)


def _ffn_kernel(be_ref, xs_ref, wg_ref, wu_ref, wd_ref, y_ref):
    b = pl.program_id(0)
    be = be_ref[b]

    @pl.when(be >= 0)
    def _():
        x = xs_ref[...]
        g = jnp.dot(x, wg_ref[0], preferred_element_type=jnp.float32)
        u = jnp.dot(x, wu_ref[0], preferred_element_type=jnp.float32)
        h = (g / (1.0 + jnp.exp(-g))) * u
        y_ref[...] = jnp.dot(h, wd_ref[0],
                             preferred_element_type=jnp.float32)


def _ffn(be, xs, w_gate, w_up, w_down):
    grid_spec = pltpu.PrefetchScalarGridSpec(
        num_scalar_prefetch=1,
        grid=(_NB,),
        in_specs=[
            pl.BlockSpec((_B, D_MODEL),
                         lambda b, be: (jnp.where(be[b] >= 0, b, 0), 0)),
            pl.BlockSpec((1, D_MODEL, D_FF),
                         lambda b, be: (jnp.maximum(be[b], 0), 0, 0)),
            pl.BlockSpec((1, D_MODEL, D_FF),
                         lambda b, be: (jnp.maximum(be[b], 0), 0, 0)),
            pl.BlockSpec((1, D_FF, D_MODEL),
                         lambda b, be: (jnp.maximum(be[b], 0), 0, 0)),
        ],
        out_specs=pl.BlockSpec((_B, D_MODEL), lambda b, be: (b, 0)),
    )
    return pl.pallas_call(
        _ffn_kernel,
        grid_spec=grid_spec,
        out_shape=jax.ShapeDtypeStruct((_P, D_MODEL), jnp.float32),
        compiler_params=pltpu.CompilerParams(
            dimension_semantics=("arbitrary",)),
    )(be, xs, w_gate, w_up, w_down)


_CH = 32


def _sc_combine_body(y_hbm, p0_hbm, p1_hbm, w0_hbm, w1_hbm, out_hbm,
                     idx0_v, idx1_v, w0_v, w1_v, buf0_v, buf1_v, sem0, sem1):
    wid = lax.axis_index("s") * 2 + lax.axis_index("c")
    base = wid * _TPW
    for ck in range(_TPW // _CH):
        off = base + ck * _CH
        la = pltpu.async_copy(p0_hbm.at[pl.ds(off, _CH)], idx0_v, sem0)
        lb = pltpu.async_copy(p1_hbm.at[pl.ds(off, _CH)], idx1_v, sem1)
        lc = pltpu.async_copy(w0_hbm.at[pl.ds(off, _CH)], w0_v, sem0)
        ld = pltpu.async_copy(w1_hbm.at[pl.ds(off, _CH)], w1_v, sem1)
        la.wait()
        lb.wait()
        lc.wait()
        ld.wait()
        cp0 = pltpu.async_copy(y_hbm.at[idx0_v], buf0_v, sem0)
        cp1 = pltpu.async_copy(y_hbm.at[idx1_v], buf1_v, sem1)
        cp0.wait()
        cp1.wait()
        for r in range(_CH):
            wv0 = w0_v[r, :]
            wv1 = w1_v[r, :]

            def _row_fma(jj, _, r=r, wv0=wv0, wv1=wv1):
                o = jj * 16
                buf0_v[r, pl.ds(o, 16)] = (
                    buf0_v[r, pl.ds(o, 16)] * wv0
                    + buf1_v[r, pl.ds(o, 16)] * wv1)
                return 0
            lax.fori_loop(0, D_MODEL // 16, _row_fma, 0, unroll=8)
        pltpu.sync_copy(buf0_v, out_hbm.at[pl.ds(off, _CH)])


def _sc_combine(y, pos0, pos1, w0, w1):
    mesh = plsc.VectorSubcoreMesh(core_axis_name="c", subcore_axis_name="s")
    kfn = functools.partial(
        pl.kernel,
        mesh=mesh,
        out_type=jax.ShapeDtypeStruct((T, D_MODEL), jnp.float32),
        scratch_types=[
            pltpu.VMEM((_CH,), jnp.int32),
            pltpu.VMEM((_CH,), jnp.int32),
            pltpu.VMEM((_CH, 16), jnp.float32),
            pltpu.VMEM((_CH, 16), jnp.float32),
            pltpu.VMEM((_CH, D_MODEL), jnp.float32),
            pltpu.VMEM((_CH, D_MODEL), jnp.float32),
            pltpu.SemaphoreType.DMA,
            pltpu.SemaphoreType.DMA,
        ],
    )(_sc_combine_body)
    return kfn(y, pos0, pos1, w0, w1)


@jax.jit
def kernel(hidden_states, gate_w, e_bias, w_gate, w_up, w_down):
    x = hidden_states.reshape(-1, D_MODEL)
    pos0, pos1, w0, w1, be = _frontend(x, gate_w, e_bias)
    p0 = pos0.reshape(T)
    p1 = pos1.reshape(T)
    xs = _sc_scatter(x, p0, p1)
    y = _ffn(be.reshape(_NB), xs, w_gate, w_up, w_down)
    return _sc_combine(y, p0, p1, w0, w1)
